# Initial kernel scaffold; baseline (speedup 1.0000x reference)
import jax
import jax.numpy as jnp
from jax.experimental import pallas as pl


def kernel(img_feat0, img_feat1, img_feat2, img_feat3, input):
    raise NotImplementedError("write your pallas kernel here")


# speedup vs baseline: 3.2309x; 3.2309x over previous
import functools

import jax
import jax.numpy as jnp
from jax import lax
from jax.experimental import pallas as pl
from jax.experimental.pallas import tpu as pltpu
from jax.experimental.pallas import tpu_sc as plsc

N_VERTS = 100000
SIZES = (56, 28, 14, 7)
CHANS = (64, 128, 256, 512)
COL_OFF = (3, 67, 195, 451)
OUT_D = 963

NUM_CORES = 2
NUM_SUBCORES = 16
NUM_WORKERS = NUM_CORES * NUM_SUBCORES
LANES = 16

CHUNK = 32
NUM_CHUNKS = N_VERTS // CHUNK
ITERS_PER_WORKER = (NUM_CHUNKS + NUM_WORKERS - 1) // NUM_WORKERS


def _make_table(feat, img_size):
    chans = feat.shape[0]
    s2 = img_size * img_size

    def body(x_ref, o_ref):
        o_ref[...] = jnp.concatenate(
            [x_ref[...].T, jnp.zeros((8, chans), jnp.float32)], axis=0
        )

    return pl.pallas_call(
        body,
        out_shape=jax.ShapeDtypeStruct((s2 + 8, chans), jnp.float32),
    )(feat.reshape(chans, s2))


def _sc_body(
    t0, t1, t2, t3, inp, out,
    inbuf, idx0, idx1, idx2, idx3, g0, g1, g2, g3, outflat, sem,
):
    wid = lax.axis_index("s") * NUM_CORES + lax.axis_index("c")
    tables = (t0, t1, t2, t3)
    idxbufs = (idx0, idx1, idx2, idx3)
    gbufs = (g0, g1, g2, g3)
    lane = lax.iota(jnp.int32, LANES)

    def chunk_body(i, _):
        cid = i * NUM_WORKERS + wid

        @pl.when(cid < NUM_CHUNKS)
        def _():
            base = cid * CHUNK
            pltpu.sync_copy(inp.at[pl.ds(base, CHUNK)], inbuf)

            for g in range(CHUNK // LANES):
                rows = lane + g * LANES
                c0 = jnp.zeros((LANES,), jnp.int32)
                in0 = plsc.load_gather(inbuf, [rows, c0])
                in1 = plsc.load_gather(inbuf, [rows, c0 + 1])
                in2 = plsc.load_gather(inbuf, [rows, c0 + 2])

                h = 248.0 * (in1 / in2) + 111.5
                w = 248.0 * (in0 / (-in2)) + 111.5
                h = jnp.minimum(jnp.maximum(h, 0.0), 223.0)
                w = jnp.minimum(jnp.maximum(w, 0.0), 223.0)

                for lvl, s in enumerate(SIZES):
                    x = h * (s / 224.0)
                    y = w * (s / 224.0)
                    xi = x.astype(jnp.int32)
                    yi = y.astype(jnp.int32)
                    x2 = jnp.minimum(
                        xi + (x > xi.astype(jnp.float32)).astype(jnp.int32), s - 1
                    )
                    y2 = jnp.minimum(
                        yi + (y > yi.astype(jnp.float32)).astype(jnp.int32), s - 1
                    )
                    ok = (x2 > xi) & (y2 > yi)
                    idx = jnp.where(ok, xi * s + yi, s * s)
                    idxbufs[lvl][pl.ds(g * LANES, LANES)] = idx

                obase = rows * OUT_D
                plsc.store_scatter(outflat, [obase], in0)
                plsc.store_scatter(outflat, [obase + 1], in1)
                plsc.store_scatter(outflat, [obase + 2], in2)

            copies = []
            for lvl in range(4):
                copies.append(
                    pltpu.async_copy(
                        tables[lvl].at[idxbufs[lvl]], gbufs[lvl], sem
                    )
                )
            for cp in copies:
                cp.wait()

            def row_body(r, _):
                rb = r * OUT_D
                for lvl in range(4):
                    gb = gbufs[lvl]
                    for k in range(CHANS[lvl] // LANES):
                        v = gb[r, pl.ds(k * LANES, LANES)]
                        dst = lane + (rb + COL_OFF[lvl] + k * LANES)
                        plsc.store_scatter(outflat, [dst], v)

            lax.fori_loop(0, CHUNK, row_body, None)

            pltpu.sync_copy(
                outflat, out.at[pl.ds(base * OUT_D, CHUNK * OUT_D)]
            )

    lax.fori_loop(0, ITERS_PER_WORKER, chunk_body, None)


@functools.partial(jax.jit, donate_argnums=())
def kernel(img_feat0, img_feat1, img_feat2, img_feat3, input):
    tables = [
        _make_table(f, s)
        for f, s in zip((img_feat0, img_feat1, img_feat2, img_feat3), SIZES)
    ]

    mesh = plsc.VectorSubcoreMesh(core_axis_name="c", subcore_axis_name="s")
    sc_call = pl.kernel(
        _sc_body,
        out_type=jax.ShapeDtypeStruct((N_VERTS * OUT_D,), jnp.float32),
        mesh=mesh,
        compiler_params=pltpu.CompilerParams(
            use_tc_tiling_on_sc=False, needs_layout_passes=False
        ),
        scratch_types=[
            pltpu.VMEM((CHUNK, 3), jnp.float32),
            pltpu.VMEM((CHUNK,), jnp.int32),
            pltpu.VMEM((CHUNK,), jnp.int32),
            pltpu.VMEM((CHUNK,), jnp.int32),
            pltpu.VMEM((CHUNK,), jnp.int32),
            pltpu.VMEM((CHUNK, CHANS[0]), jnp.float32),
            pltpu.VMEM((CHUNK, CHANS[1]), jnp.float32),
            pltpu.VMEM((CHUNK, CHANS[2]), jnp.float32),
            pltpu.VMEM((CHUNK, CHANS[3]), jnp.float32),
            pltpu.VMEM((CHUNK * OUT_D,), jnp.float32),
            pltpu.SemaphoreType.DMA,
        ],
    )
    return sc_call(*tables, input).reshape(N_VERTS, OUT_D)
